# baseline (speedup 1.0000x reference)
import functools

import jax
import jax.numpy as jnp
from jax import lax
from jax.experimental import pallas as pl
from jax.experimental.pallas import tpu as pltpu
from jax.experimental.pallas import tpu_sc as plsc

_B, _L, _V, _D = 1024, 50, 1000, 64
_N = _B * _L
_ROWS = 512
_NB = _N // _ROWS
_NW = 32
_TPW = _N // _NW
_CHUNK = 80
_NCH = _TPW // _CHUNK


def _argmax_body(x_ref, o_ref):
    x = x_ref[...]
    col = lax.broadcasted_iota(jnp.int32, x.shape, 1)
    mx = jnp.max(x, axis=1, keepdims=True)
    cand = jnp.where(x == mx, col, _V)
    idx = jnp.min(cand, axis=1)
    o_ref[...] = idx.reshape(1, 1, _ROWS)


def _argmax_tc(iw2):
    return pl.pallas_call(
        _argmax_body,
        grid=(_NB,),
        in_specs=[pl.BlockSpec((_ROWS, _V), lambda i: (i, 0))],
        out_specs=pl.BlockSpec((1, 1, _ROWS), lambda i: (i, 0, 0)),
        out_shape=jax.ShapeDtypeStruct((_NB, 1, _ROWS), jnp.int32),
    )(iw2)


def _sc_lookup(table, idx3, pos):
    mesh = plsc.VectorSubcoreMesh(core_axis_name="c", subcore_axis_name="s")

    @functools.partial(
        pl.kernel,
        out_type=jax.ShapeDtypeStruct((_N, _D), jnp.float32),
        mesh=mesh,
        scratch_types=[
            pltpu.VMEM((_NCH, _CHUNK), jnp.int32),
            pltpu.VMEM((_TPW, _D), jnp.float32),
            pltpu.SemaphoreType.DMA,
            pltpu.SemaphoreType.DMA,
        ],
        compiler_params=pltpu.CompilerParams(use_tc_tiling_on_sc=False),
    )
    def k(table_hbm, idx_hbm, pos_hbm, out_hbm, idx_v, dest, sem_p, sem_g):
        wid = lax.axis_index("s") * 2 + lax.axis_index("c")
        base = wid * _TPW
        pltpu.sync_copy(idx_hbm.at[wid], idx_v)
        pos_cps = [
            pltpu.async_copy(pos_hbm, dest.at[pl.ds(i * _L, _L)], sem_p)
            for i in range(_TPW // _L)
        ]
        for c in pos_cps:
            c.wait()
        g_cps = [
            pltpu.async_copy(
                table_hbm.at[idx_v.at[j]],
                dest.at[pl.ds(j * _CHUNK, _CHUNK)],
                sem_g,
                add=True,
            )
            for j in range(_NCH)
        ]
        for c in g_cps:
            c.wait()
        pltpu.sync_copy(dest, out_hbm.at[pl.ds(base, _TPW)])

    return k(table, idx3, pos)


def kernel(index_weights, start_pos, emb_table, pos_emb):
    iw2 = index_weights.reshape(_N, _V)
    idx = _argmax_tc(iw2)
    idx3 = idx.reshape(_NW, _NCH, _CHUNK)
    pos = lax.dynamic_slice_in_dim(pos_emb, start_pos, _L, axis=0)
    out = _sc_lookup(emb_table, idx3, pos)
    return out.reshape(_B, _L, _D)


# speedup vs baseline: 3.2801x; 3.2801x over previous
import functools

import jax
import jax.numpy as jnp
from jax import lax
from jax.experimental import pallas as pl
from jax.experimental.pallas import tpu as pltpu
from jax.experimental.pallas import tpu_sc as plsc

_B, _L, _V, _D = 1024, 50, 1000, 64
_N = _B * _L
_LANES = 128
_SUB = _B // _LANES
_NCHUNK = _L * _SUB
_NW = 32
_CPW = 13
_TPW = _CPW * _LANES


def _argmax_body(x_ref, o_ref):
    for k in range(_SUB):
        x = x_ref[0, :, k * _LANES:(k + 1) * _LANES]
        row = lax.broadcasted_iota(jnp.int32, x.shape, 0)
        mx = jnp.max(x, axis=0, keepdims=True)
        cand = jnp.where(x == mx, row, _V)
        o_ref[0, k, :] = jnp.min(cand, axis=0)


def _argmax_tc(iw_t):
    return pl.pallas_call(
        _argmax_body,
        grid=(_L,),
        in_specs=[pl.BlockSpec((1, _V, _B), lambda i: (i, 0, 0))],
        out_specs=pl.BlockSpec((1, _SUB, _LANES), lambda i: (i, 0, 0)),
        out_shape=jax.ShapeDtypeStruct((_L, _SUB, _LANES), jnp.int32),
    )(iw_t)


def _sc_lookup(table, idx2, posb):
    mesh = plsc.VectorSubcoreMesh(core_axis_name="c", subcore_axis_name="s")

    @functools.partial(
        pl.kernel,
        out_type=jax.ShapeDtypeStruct((_N, _D), jnp.float32),
        mesh=mesh,
        scratch_types=[
            pltpu.VMEM((_CPW, _LANES), jnp.int32),
            pltpu.VMEM((_TPW, _D), jnp.float32),
            pltpu.SemaphoreType.DMA,
            pltpu.SemaphoreType.DMA,
        ],
        compiler_params=pltpu.CompilerParams(use_tc_tiling_on_sc=False),
    )
    def k(table_hbm, idx_hbm, posb_hbm, out_hbm, idx_v, dest, sem_p, sem_g):
        wid = lax.axis_index("s") * 2 + lax.axis_index("c")
        cbase = (25 * wid) // 2
        pltpu.sync_copy(idx_hbm.at[pl.ds(cbase, _CPW)], idx_v)
        pos_cps = [
            pltpu.async_copy(
                posb_hbm.at[pl.ds(((cbase + j) // _SUB) * _LANES, _LANES)],
                dest.at[pl.ds(j * _LANES, _LANES)],
                sem_p,
            )
            for j in range(_CPW)
        ]
        for c in pos_cps:
            c.wait()
        g_cps = [
            pltpu.async_copy(
                table_hbm.at[idx_v.at[j]],
                dest.at[pl.ds(j * _LANES, _LANES)],
                sem_g,
                add=True,
            )
            for j in range(_CPW)
        ]
        for c in g_cps:
            c.wait()
        pltpu.sync_copy(dest, out_hbm.at[pl.ds(cbase * _LANES, _TPW)])

    return k(table, idx2, posb)


def kernel(index_weights, start_pos, emb_table, pos_emb):
    iw_t = jnp.transpose(index_weights, (1, 2, 0))
    idx = _argmax_tc(iw_t)
    idx2 = idx.reshape(_NCHUNK, _LANES)
    pos = lax.dynamic_slice_in_dim(pos_emb, start_pos, _L, axis=0)
    posb = jnp.broadcast_to(pos[:, None, :], (_L, _LANES, _D)).reshape(
        _L * _LANES, _D)
    out_t = _sc_lookup(emb_table, idx2, posb)
    return out_t.reshape(_L, _B, _D).transpose(1, 0, 2)
